# baseline (speedup 1.0000x reference)
import jax
import jax.numpy as jnp
from jax import lax
from jax.experimental import pallas as pl
from jax.experimental.pallas import tpu as pltpu
from jax.experimental.pallas import tpu_sc as plsc

N = 10000
E = 320000
DIN = 128
H = 256
HH = 128
G = 64
DOUT = 4
NP = 10240
NT = 16
NC = 2
K = 80
NBUF = 2
CH_P = E // NT // K
CH_D = E // (NC * NT) // K
NH = N // 2
DUMMY = NH
RPH = 312
HTAIL0 = RPH * NT
HTAILN = NH - HTAIL0
RPTP = NP // NT
RB = 1000
NB = N // RB


def _sc_mesh():
    return plsc.VectorSubcoreMesh(core_axis_name="c", subcore_axis_name="s")


def _deg_body(dst_hbm, out_hbm, idxbuf, ones_v, zeros_v, acc):
    c = lax.axis_index("c")
    s = lax.axis_index("s")

    def fz(i, _):
        zeros_v[pl.ds(i * 16, 16)] = jnp.zeros((16,), jnp.float32)
        return 0
    lax.fori_loop(0, RPTP // 16, fz, 0)

    def fo(i, _):
        ones_v[pl.ds(i * 16, 16)] = jnp.ones((16,), jnp.float32)
        return 0
    lax.fori_loop(0, K // 16, fo, 0)

    pltpu.sync_copy(zeros_v, acc.at[pl.ds(s * RPTP, RPTP)])
    pltpu.sync_copy(dst_hbm.at[c, s], idxbuf)
    plsc.subcore_barrier()

    def body(i, _):
        pltpu.sync_copy(ones_v, acc.at[idxbuf.at[i]], add=True)
        return 0
    lax.fori_loop(0, CH_D, body, 0)

    plsc.subcore_barrier()
    pltpu.sync_copy(acc.at[pl.ds(s * RPTP, RPTP)],
                    out_hbm.at[c, pl.ds(s * RPTP, RPTP)])


def _deg_call(dst_r):
    f = pl.kernel(
        _deg_body,
        out_type=jax.ShapeDtypeStruct((NC, NP), jnp.float32),
        mesh=_sc_mesh(),
        scratch_types=[
            pltpu.VMEM((CH_D, K), jnp.int32),
            pltpu.VMEM((K,), jnp.float32),
            pltpu.VMEM((RPTP,), jnp.float32),
            pltpu.VMEM_SHARED((NP,), jnp.float32),
        ],
    )
    return f(dst_r)


def _prop_body(zf0, zf1, zi0, zi1, lo_hbm, nch_hbm, src_r, dst_r, out0, out1,
               sidx, didx, r0, r1, lobuf, nbuf, acc, g0s, g1s, s0s, s1s):
    c = lax.axis_index("c")
    s = lax.axis_index("s")
    rowsb = (r0, r1)
    gsems = (g0s, g1s)
    ssems = (s0s, s1s)
    pltpu.sync_copy(lo_hbm, lobuf)
    pltpu.sync_copy(nch_hbm, nbuf)
    lo = lobuf[...]
    n = nbuf[...][0]
    for cc in (0, 1):
        zf = (zf0, zf1)[cc]
        zi = (zi0, zi1)[cc]
        out = (out0, out1)[cc]

        @pl.when(c == cc)
        def _(zf=zf, zi=zi, out=out):
            pltpu.sync_copy(zi.at[pl.ds(s * RPH, RPH)],
                            acc.at[pl.ds(s * RPH, RPH)])

            @pl.when(s == NT - 1)
            def _(zi=zi):
                pltpu.sync_copy(zi.at[pl.ds(HTAIL0, HTAILN)],
                                acc.at[pl.ds(HTAIL0, HTAILN)])

            pltpu.sync_copy(src_r.at[s], sidx)
            pltpu.sync_copy(dst_r.at[s], didx)
            plsc.subcore_barrier()

            def transform(i):
                for u in range(K // 16):
                    d = didx[i, pl.ds(u * 16, 16)]
                    m = (d >= lo) & (d < lo + NH)
                    didx[i, pl.ds(u * 16, 16)] = jnp.where(
                        m, d - lo, DUMMY + (d & 7))

            def start_gather(i, b, zf=zf):
                pltpu.async_copy(zf.at[sidx.at[i]], rowsb[b], gsems[b])

            def wait_gather(i, b, zf=zf):
                pltpu.make_async_copy(zf.at[sidx.at[i]], rowsb[b],
                                      gsems[b]).wait()

            def start_scatter(i, b):
                pltpu.async_copy(rowsb[b], acc.at[didx.at[i]], ssems[b],
                                 add=True)

            def wait_scatter(i, b):
                pltpu.make_async_copy(rowsb[b], acc.at[didx.at[i]],
                                      ssems[b]).wait()

            @pl.when(n > 0)
            def _():
                start_gather(0, 0)

            def body(j, _):
                for u in range(NBUF):
                    i = NBUF * j + u
                    b = u
                    transform(i)
                    wait_gather(i, b)
                    start_scatter(i, b)
                    ip = i + NBUF - 1
                    bp = (u + NBUF - 1) % NBUF

                    @pl.when(ip < n)
                    def _(i=i, ip=ip, bp=bp):
                        @pl.when(i >= 1)
                        def _():
                            wait_scatter(i - 1, bp)
                        start_gather(ip, bp)
                return 0
            lax.fori_loop(0, n // NBUF, body, 0)

            @pl.when(n > 0)
            def _():
                for u in range(NBUF):
                    wait_scatter(n - NBUF + u, u)

            plsc.subcore_barrier()
            pltpu.sync_copy(acc.at[pl.ds(s * RPH, RPH)],
                            out.at[pl.ds(s * RPH, RPH)])

            @pl.when(s == NT - 1)
            def _(out=out):
                pltpu.sync_copy(acc.at[pl.ds(HTAIL0, HTAILN)],
                                out.at[pl.ds(HTAIL0, HTAILN)])


def _prop_call(zf0, zf1, zi0, zi1, lov, nchv, src_r, dst_r):
    f = pl.kernel(
        _prop_body,
        out_type=(jax.ShapeDtypeStruct((NH, HH), jnp.float32),
                  jax.ShapeDtypeStruct((NH, HH), jnp.float32)),
        mesh=_sc_mesh(),
        scratch_types=(
            [pltpu.VMEM((CH_P, K), jnp.int32),
             pltpu.VMEM((CH_P, K), jnp.int32)]
            + [pltpu.VMEM((K, HH), jnp.float32) for _ in range(NBUF)]
            + [pltpu.VMEM((16,), jnp.int32),
               pltpu.VMEM((16,), jnp.int32),
               pltpu.VMEM_SHARED((NH + 8, HH), jnp.float32)]
            + [pltpu.SemaphoreType.DMA for _ in range(2 * NBUF)]
        ),
    )
    return f(zf0, zf1, zi0, zi1, lov, nchv, src_r, dst_r)


def _dinv_of(deg_blk):
    deg = deg_blk[:, 0] + deg_blk[:, 1] + 1.0
    return lax.rsqrt(jnp.maximum(deg, 1e-12))


def _unhalf(o0_ref, o1_ref):
    return jnp.concatenate([o0_ref[...], o1_ref[...]], axis=1)


def _mm0_body(deg_ref, x_ref, w_ref, za_ref, zb_ref):
    dinv = _dinv_of(deg_ref[...])
    z = jnp.dot(x_ref[...], w_ref[...],
                preferred_element_type=jnp.float32) * dinv[:, None]
    za_ref[...] = z[:, :HH]
    zb_ref[...] = z[:, HH:]


def _mm0_call(deg2, x, w):
    return pl.pallas_call(
        _mm0_body,
        grid=(NB,),
        in_specs=[
            pl.BlockSpec((RB, NC), lambda i: (i, 0)),
            pl.BlockSpec((RB, DIN), lambda i: (i, 0)),
            pl.BlockSpec((DIN, H), lambda i: (0, 0)),
        ],
        out_specs=[
            pl.BlockSpec((RB, HH), lambda i: (i, 0)),
            pl.BlockSpec((RB, HH), lambda i: (i, 0)),
        ],
        out_shape=[jax.ShapeDtypeStruct((N, HH), jnp.float32),
                   jax.ShapeDtypeStruct((N, HH), jnp.float32)],
    )(deg2, x, w)


def _half_spec():
    return pl.BlockSpec((RB, HH), lambda i: (i, 0))


def _stats_body(deg_ref, o0_ref, o1_ref, o_ref):
    i = pl.program_id(0)
    dinv = _dinv_of(deg_ref[...])
    t = _unhalf(o0_ref, o1_ref) * dinv[:, None]
    part = jnp.stack([jnp.sum(t, axis=0), jnp.sum(t * t, axis=0)])

    @pl.when(i == 0)
    def _():
        o_ref[...] = part

    @pl.when(i > 0)
    def _():
        o_ref[...] = o_ref[...] + part


def _stats_call(deg2, o0s, o1s):
    return pl.pallas_call(
        _stats_body,
        grid=(NB,),
        in_specs=[
            pl.BlockSpec((RB, NC), lambda i: (i, 0)),
            _half_spec(),
            _half_spec(),
        ],
        out_specs=pl.BlockSpec((2, H), lambda i: (0, 0)),
        out_shape=jax.ShapeDtypeStruct((2, H), jnp.float32),
    )(deg2, o0s, o1s)


def _bnmm_body(deg_ref, o0_ref, o1_ref, stats_ref, g_ref, be_ref, w_ref,
               za_ref, zb_ref, h_ref):
    dinv = _dinv_of(deg_ref[...])
    t = _unhalf(o0_ref, o1_ref) * dinv[:, None]
    mean = stats_ref[0] / float(N)
    var = stats_ref[1] / float(N) - mean * mean
    h = (t - mean) * lax.rsqrt(var + 1e-5) * g_ref[...] + be_ref[...]
    h = jnp.maximum(h, 0.0)
    z = jnp.dot(h, w_ref[...], preferred_element_type=jnp.float32) * dinv[:, None]
    za_ref[...] = z[:, :HH]
    zb_ref[...] = z[:, HH:]
    h_ref[...] = h


def _bnmm_call(deg2, o0s, o1s, stats, g, be, w):
    return pl.pallas_call(
        _bnmm_body,
        grid=(NB,),
        in_specs=[
            pl.BlockSpec((RB, NC), lambda i: (i, 0)),
            _half_spec(),
            _half_spec(),
            pl.BlockSpec((2, H), lambda i: (0, 0)),
            pl.BlockSpec((H,), lambda i: (0,)),
            pl.BlockSpec((H,), lambda i: (0,)),
            pl.BlockSpec((H, H), lambda i: (0, 0)),
        ],
        out_specs=[
            pl.BlockSpec((RB, HH), lambda i: (i, 0)),
            pl.BlockSpec((RB, HH), lambda i: (i, 0)),
            pl.BlockSpec((RB, H), lambda i: (i, 0)),
        ],
        out_shape=[jax.ShapeDtypeStruct((N, HH), jnp.float32),
                   jax.ShapeDtypeStruct((N, HH), jnp.float32),
                   jax.ShapeDtypeStruct((N, H), jnp.float32)],
    )(deg2, o0s, o1s, stats, g, be, w)


def _pool_body(h_ref, bs_ref, bv_ref, zsum_ref, zmax_ref, cnt_ref):
    i = pl.program_id(0)

    @pl.when(i == 0)
    def _():
        zsum_ref[...] = jnp.zeros_like(zsum_ref)
        zmax_ref[...] = jnp.full_like(zmax_ref, -jnp.inf)
        cnt_ref[...] = jnp.zeros_like(cnt_ref)

    h = h_ref[...]
    bv = bv_ref[...]
    glo = bs_ref[0, 0, 0]
    ghi = bs_ref[0, 0, RB - 1]

    def body(gid, _):
        m = bv == gid
        hmax = jnp.max(jnp.where(m, h, -jnp.inf), axis=0)
        hsum = jnp.sum(jnp.where(m, h, 0.0), axis=0)
        cnt = jnp.sum(m.astype(jnp.float32))
        zmax_ref[pl.ds(gid, 1), :] = jnp.maximum(zmax_ref[pl.ds(gid, 1), :],
                                                 hmax[None])
        zsum_ref[pl.ds(gid, 1), :] = zsum_ref[pl.ds(gid, 1), :] + hsum[None]
        cnt_ref[pl.ds(gid, 1), :] = cnt_ref[pl.ds(gid, 1), :] + cnt
        return 0
    lax.fori_loop(glo, ghi + 1, body, 0)


def _pool_call(h, batch2, batchc):
    return pl.pallas_call(
        _pool_body,
        grid=(NB,),
        in_specs=[
            pl.BlockSpec((RB, H), lambda i: (i, 0)),
            pl.BlockSpec((1, 1, RB), lambda i: (i, 0, 0), memory_space=pltpu.SMEM),
            pl.BlockSpec((RB, 1), lambda i: (i, 0)),
        ],
        out_specs=[
            pl.BlockSpec((G, H), lambda i: (0, 0)),
            pl.BlockSpec((G, H), lambda i: (0, 0)),
            pl.BlockSpec((G, 128), lambda i: (0, 0)),
        ],
        out_shape=[jax.ShapeDtypeStruct((G, H), jnp.float32),
                   jax.ShapeDtypeStruct((G, H), jnp.float32),
                   jax.ShapeDtypeStruct((G, 128), jnp.float32)],
    )(h, batch2, batchc)


def _mlp_body(zsum_ref, zmax_ref, cnt_ref, w1_ref, b1_ref, w2_ref, b2_ref,
              o_ref):
    cnt = jnp.maximum(cnt_ref[:, :1], 1.0)
    z = jnp.concatenate([zsum_ref[...] / cnt, zmax_ref[...]], axis=1)
    y = jnp.dot(z, w1_ref[...], preferred_element_type=jnp.float32) + b1_ref[...]
    y = jnp.maximum(y, 0.0)
    o_ref[...] = jnp.dot(y, w2_ref[...],
                         preferred_element_type=jnp.float32) + b2_ref[...]


def _mlp_call(zsum, zmax, cnt, w1, b1, w2, b2):
    return pl.pallas_call(
        _mlp_body,
        out_shape=jax.ShapeDtypeStruct((G, DOUT), jnp.float32),
    )(zsum, zmax, cnt, w1, b1, w2, b2)


def kernel(x, edge_index, batch, W0, b0, g0, be0, W1, b1, g1, be1,
           W2, b2, g2, be2, fc1_W, fc1_b, fc2_W, fc2_b):
    src = edge_index[0]
    dst = edge_index[1]
    dst_deg = dst.reshape(NC, NT, CH_D, K)
    batch2 = batch.reshape(NB, 1, RB)
    batchc = batch.reshape(N, 1)
    lo_tab = jnp.array([[0] * 16, [NH] * 16], jnp.int32)

    key = (dst >= NH).astype(jnp.int32)
    pos0 = jnp.cumsum(1 - key) - 1
    pos1 = jnp.cumsum(key) - 1
    e0 = pos0[-1] + 1
    e1 = E - e0
    CAP = NT * CH_P * K
    PU = NT * K * NBUF

    def _lay(slot):
        cg = slot // K
        return ((cg % NT) * CH_P + cg // NT) * K + slot % K

    p0 = jnp.where(key == 0, _lay(pos0), CAP)
    p1 = jnp.where(key == 1, _lay(pos1), CAP)
    srch = jnp.stack([
        jnp.zeros((CAP,), jnp.int32).at[p0].set(src, mode="drop"),
        jnp.zeros((CAP,), jnp.int32).at[p1].set(src, mode="drop"),
    ]).reshape(2, NT, CH_P, K)
    dsth = jnp.stack([
        jnp.full((CAP,), N, jnp.int32).at[p0].set(dst, mode="drop"),
        jnp.full((CAP,), N, jnp.int32).at[p1].set(dst, mode="drop"),
    ]).reshape(2, NT, CH_P, K)
    nch0 = NBUF * ((e0 + PU - 1) // PU)
    nch1 = NBUF * ((e1 + PU - 1) // PU)
    nch_tab = jnp.stack([jnp.full((16,), nch0, jnp.int32),
                         jnp.full((16,), nch1, jnp.int32)])

    deg2 = _deg_call(dst_deg).T

    za, zb = _mm0_call(deg2, x, W0)

    w_stack = jnp.stack([W1, W2, jnp.eye(H, dtype=jnp.float32)])
    g_stack = jnp.stack([g0, g1, g2])
    be_stack = jnp.stack([be0, be1, be2])

    def _layer(carry, wgb):
        za, zb, _ = carry
        w, g, be = wgb

        def _hprop(_, xs):
            zi0, zi1, lov, nchv, sr, dr = xs
            o0, o1 = _prop_call(za, zb, zi0, zi1, lov, nchv, sr, dr)
            return 0, (o0, o1)

        _, (o0s, o1s) = lax.scan(
            _hprop, 0,
            (za.reshape(2, NH, HH), zb.reshape(2, NH, HH), lo_tab,
             nch_tab, srch, dsth))
        sa = o0s.reshape(N, HH)
        sb = o1s.reshape(N, HH)
        stats = _stats_call(deg2, sa, sb)
        za2, zb2, h = _bnmm_call(deg2, sa, sb, stats, g, be, w)
        return (za2, zb2, h), None

    zero_h = jnp.zeros((N, H), jnp.float32)
    (_, _, h), _ = lax.scan(
        _layer, (za, zb, zero_h), (w_stack, g_stack, be_stack))

    zsum, zmax, cnt = _pool_call(h, batch2, batchc)
    return _mlp_call(zsum, zmax, cnt, fc1_W, fc1_b, fc2_W, fc2_b)


# speedup vs baseline: 1.0001x; 1.0001x over previous
import jax
import jax.numpy as jnp
from jax import lax
from jax.experimental import pallas as pl
from jax.experimental.pallas import tpu as pltpu
from jax.experimental.pallas import tpu_sc as plsc

N = 10000
E = 320000
DIN = 128
H = 256
HH = 128
G = 64
DOUT = 4
NP = 10240
NT = 16
NC = 2
K = 80
NBUF = 2
CH_P = E // NT // K
CH_D = E // (NC * NT) // K
NH = N // 2
DUMMY = NH
RPH = 312
HTAIL0 = RPH * NT
HTAILN = NH - HTAIL0
RPTP = NP // NT
RB = 1000
NB = N // RB


def _sc_mesh():
    return plsc.VectorSubcoreMesh(core_axis_name="c", subcore_axis_name="s")


def _deg_body(dst_hbm, out_hbm, idxbuf, ones_v, zeros_v, acc):
    c = lax.axis_index("c")
    s = lax.axis_index("s")

    def fz(i, _):
        zeros_v[pl.ds(i * 16, 16)] = jnp.zeros((16,), jnp.float32)
        return 0
    lax.fori_loop(0, RPTP // 16, fz, 0)

    def fo(i, _):
        ones_v[pl.ds(i * 16, 16)] = jnp.ones((16,), jnp.float32)
        return 0
    lax.fori_loop(0, K // 16, fo, 0)

    pltpu.sync_copy(zeros_v, acc.at[pl.ds(s * RPTP, RPTP)])
    pltpu.sync_copy(dst_hbm.at[c, s], idxbuf)
    plsc.subcore_barrier()

    def body(i, _):
        pltpu.sync_copy(ones_v, acc.at[idxbuf.at[i]], add=True)
        return 0
    lax.fori_loop(0, CH_D, body, 0)

    plsc.subcore_barrier()
    pltpu.sync_copy(acc.at[pl.ds(s * RPTP, RPTP)],
                    out_hbm.at[c, pl.ds(s * RPTP, RPTP)])


def _deg_call(dst_r):
    f = pl.kernel(
        _deg_body,
        out_type=jax.ShapeDtypeStruct((NC, NP), jnp.float32),
        mesh=_sc_mesh(),
        scratch_types=[
            pltpu.VMEM((CH_D, K), jnp.int32),
            pltpu.VMEM((K,), jnp.float32),
            pltpu.VMEM((RPTP,), jnp.float32),
            pltpu.VMEM_SHARED((NP,), jnp.float32),
        ],
    )
    return f(dst_r)


def _prop_body(zf0, zf1, zi0, zi1, lo_hbm, nch_hbm, src_r, dst_r, out0, out1,
               sidx, didx, r0, r1, lobuf, nbuf, acc, g0s, g1s, s0s, s1s):
    c = lax.axis_index("c")
    s = lax.axis_index("s")
    rowsb = (r0, r1)
    gsems = (g0s, g1s)
    ssems = (s0s, s1s)
    pltpu.sync_copy(lo_hbm, lobuf)
    pltpu.sync_copy(nch_hbm, nbuf)
    lo = lobuf[...]
    n = nbuf[...][0]
    for cc in (0, 1):
        zf = (zf0, zf1)[cc]
        zi = (zi0, zi1)[cc]
        out = (out0, out1)[cc]

        @pl.when(c == cc)
        def _(zf=zf, zi=zi, out=out):
            pltpu.sync_copy(zi.at[pl.ds(s * RPH, RPH)],
                            acc.at[pl.ds(s * RPH, RPH)])

            @pl.when(s == NT - 1)
            def _(zi=zi):
                pltpu.sync_copy(zi.at[pl.ds(HTAIL0, HTAILN)],
                                acc.at[pl.ds(HTAIL0, HTAILN)])

            pltpu.sync_copy(src_r.at[s], sidx)
            pltpu.sync_copy(dst_r.at[s], didx)
            plsc.subcore_barrier()

            def transform(i):
                for u in range(K // 16):
                    d = didx[i, pl.ds(u * 16, 16)]
                    m = (d >= lo) & (d < lo + NH)
                    didx[i, pl.ds(u * 16, 16)] = jnp.where(
                        m, d - lo, DUMMY + (d & 7))

            def start_gather(i, b, zf=zf):
                pltpu.async_copy(zf.at[sidx.at[i]], rowsb[b], gsems[b])

            def wait_gather(i, b, zf=zf):
                pltpu.make_async_copy(zf.at[sidx.at[i]], rowsb[b],
                                      gsems[b]).wait()

            def start_scatter(i, b):
                pltpu.async_copy(rowsb[b], acc.at[didx.at[i]], ssems[b],
                                 add=True)

            def wait_scatter(i, b):
                pltpu.make_async_copy(rowsb[b], acc.at[didx.at[i]],
                                      ssems[b]).wait()

            @pl.when(n > 0)
            def _():
                start_gather(0, 0)

            def body(j, _):
                for u in range(NBUF):
                    i = NBUF * j + u
                    b = u
                    transform(i)
                    wait_gather(i, b)
                    start_scatter(i, b)
                    ip = i + NBUF - 1
                    bp = (u + NBUF - 1) % NBUF

                    @pl.when(ip < n)
                    def _(i=i, ip=ip, bp=bp):
                        @pl.when(i >= 1)
                        def _():
                            wait_scatter(i - 1, bp)
                        start_gather(ip, bp)
                return 0
            lax.fori_loop(0, n // NBUF, body, 0)

            @pl.when(n > 0)
            def _():
                for u in range(NBUF):
                    wait_scatter(n - NBUF + u, u)

            plsc.subcore_barrier()
            pltpu.sync_copy(acc.at[pl.ds(s * RPH, RPH)],
                            out.at[pl.ds(s * RPH, RPH)])

            @pl.when(s == NT - 1)
            def _(out=out):
                pltpu.sync_copy(acc.at[pl.ds(HTAIL0, HTAILN)],
                                out.at[pl.ds(HTAIL0, HTAILN)])


def _prop_call(zf0, zf1, zi0, zi1, lov, nchv, src_r, dst_r):
    f = pl.kernel(
        _prop_body,
        out_type=(jax.ShapeDtypeStruct((NH, HH), jnp.float32),
                  jax.ShapeDtypeStruct((NH, HH), jnp.float32)),
        mesh=_sc_mesh(),
        scratch_types=(
            [pltpu.VMEM((CH_P, K), jnp.int32),
             pltpu.VMEM((CH_P, K), jnp.int32)]
            + [pltpu.VMEM((K, HH), jnp.float32) for _ in range(NBUF)]
            + [pltpu.VMEM((16,), jnp.int32),
               pltpu.VMEM((16,), jnp.int32),
               pltpu.VMEM_SHARED((NH + 8, HH), jnp.float32)]
            + [pltpu.SemaphoreType.DMA for _ in range(2 * NBUF)]
        ),
    )
    return f(zf0, zf1, zi0, zi1, lov, nchv, src_r, dst_r)


def _dinv_of(deg_blk):
    deg = deg_blk[:, 0] + deg_blk[:, 1] + 1.0
    return lax.rsqrt(jnp.maximum(deg, 1e-12))


def _unhalf(o0_ref, o1_ref):
    return jnp.concatenate([o0_ref[...], o1_ref[...]], axis=1)


def _mm0_body(deg_ref, x_ref, w_ref, za_ref, zb_ref):
    dinv = _dinv_of(deg_ref[...])
    z = jnp.dot(x_ref[...], w_ref[...],
                preferred_element_type=jnp.float32) * dinv[:, None]
    za_ref[...] = z[:, :HH]
    zb_ref[...] = z[:, HH:]


def _mm0_call(deg2, x, w):
    return pl.pallas_call(
        _mm0_body,
        grid=(NB,),
        in_specs=[
            pl.BlockSpec((RB, NC), lambda i: (i, 0)),
            pl.BlockSpec((RB, DIN), lambda i: (i, 0)),
            pl.BlockSpec((DIN, H), lambda i: (0, 0)),
        ],
        out_specs=[
            pl.BlockSpec((RB, HH), lambda i: (i, 0)),
            pl.BlockSpec((RB, HH), lambda i: (i, 0)),
        ],
        out_shape=[jax.ShapeDtypeStruct((N, HH), jnp.float32),
                   jax.ShapeDtypeStruct((N, HH), jnp.float32)],
    )(deg2, x, w)


def _half_spec():
    return pl.BlockSpec((RB, HH), lambda i: (i, 0))


def _stats_body(deg_ref, o0_ref, o1_ref, o_ref):
    i = pl.program_id(0)
    dinv = _dinv_of(deg_ref[...])
    t = _unhalf(o0_ref, o1_ref) * dinv[:, None]
    part = jnp.stack([jnp.sum(t, axis=0), jnp.sum(t * t, axis=0)])

    @pl.when(i == 0)
    def _():
        o_ref[...] = part

    @pl.when(i > 0)
    def _():
        o_ref[...] = o_ref[...] + part


def _stats_call(deg2, o0s, o1s):
    return pl.pallas_call(
        _stats_body,
        grid=(NB,),
        in_specs=[
            pl.BlockSpec((RB, NC), lambda i: (i, 0)),
            _half_spec(),
            _half_spec(),
        ],
        out_specs=pl.BlockSpec((2, H), lambda i: (0, 0)),
        out_shape=jax.ShapeDtypeStruct((2, H), jnp.float32),
    )(deg2, o0s, o1s)


def _bnmm_body(deg_ref, o0_ref, o1_ref, stats_ref, g_ref, be_ref, w_ref,
               za_ref, zb_ref, h_ref):
    dinv = _dinv_of(deg_ref[...])
    t = _unhalf(o0_ref, o1_ref) * dinv[:, None]
    mean = stats_ref[0] / float(N)
    var = stats_ref[1] / float(N) - mean * mean
    h = (t - mean) * lax.rsqrt(var + 1e-5) * g_ref[...] + be_ref[...]
    h = jnp.maximum(h, 0.0)
    z = jnp.dot(h, w_ref[...], preferred_element_type=jnp.float32) * dinv[:, None]
    za_ref[...] = z[:, :HH]
    zb_ref[...] = z[:, HH:]
    h_ref[...] = h


def _bnmm_call(deg2, o0s, o1s, stats, g, be, w):
    return pl.pallas_call(
        _bnmm_body,
        grid=(NB,),
        in_specs=[
            pl.BlockSpec((RB, NC), lambda i: (i, 0)),
            _half_spec(),
            _half_spec(),
            pl.BlockSpec((2, H), lambda i: (0, 0)),
            pl.BlockSpec((H,), lambda i: (0,)),
            pl.BlockSpec((H,), lambda i: (0,)),
            pl.BlockSpec((H, H), lambda i: (0, 0)),
        ],
        out_specs=[
            pl.BlockSpec((RB, HH), lambda i: (i, 0)),
            pl.BlockSpec((RB, HH), lambda i: (i, 0)),
            pl.BlockSpec((RB, H), lambda i: (i, 0)),
        ],
        out_shape=[jax.ShapeDtypeStruct((N, HH), jnp.float32),
                   jax.ShapeDtypeStruct((N, HH), jnp.float32),
                   jax.ShapeDtypeStruct((N, H), jnp.float32)],
    )(deg2, o0s, o1s, stats, g, be, w)


def _pool_body(h_ref, bs_ref, bv_ref, zsum_ref, zmax_ref, cnt_ref):
    i = pl.program_id(0)

    @pl.when(i == 0)
    def _():
        zsum_ref[...] = jnp.zeros_like(zsum_ref)
        zmax_ref[...] = jnp.full_like(zmax_ref, -jnp.inf)
        cnt_ref[...] = jnp.zeros_like(cnt_ref)

    h = h_ref[...]
    bv = bv_ref[...]
    glo = bs_ref[0, 0, 0]
    ghi = bs_ref[0, 0, RB - 1]

    def body(gid, _):
        m = bv == gid
        hmax = jnp.max(jnp.where(m, h, -jnp.inf), axis=0)
        hsum = jnp.sum(jnp.where(m, h, 0.0), axis=0)
        cnt = jnp.sum(m.astype(jnp.float32))
        zmax_ref[pl.ds(gid, 1), :] = jnp.maximum(zmax_ref[pl.ds(gid, 1), :],
                                                 hmax[None])
        zsum_ref[pl.ds(gid, 1), :] = zsum_ref[pl.ds(gid, 1), :] + hsum[None]
        cnt_ref[pl.ds(gid, 1), :] = cnt_ref[pl.ds(gid, 1), :] + cnt
        return 0
    lax.fori_loop(glo, ghi + 1, body, 0)


def _pool_call(h, batch2, batchc):
    return pl.pallas_call(
        _pool_body,
        grid=(NB,),
        in_specs=[
            pl.BlockSpec((RB, H), lambda i: (i, 0)),
            pl.BlockSpec((1, 1, RB), lambda i: (i, 0, 0), memory_space=pltpu.SMEM),
            pl.BlockSpec((RB, 1), lambda i: (i, 0)),
        ],
        out_specs=[
            pl.BlockSpec((G, H), lambda i: (0, 0)),
            pl.BlockSpec((G, H), lambda i: (0, 0)),
            pl.BlockSpec((G, 128), lambda i: (0, 0)),
        ],
        out_shape=[jax.ShapeDtypeStruct((G, H), jnp.float32),
                   jax.ShapeDtypeStruct((G, H), jnp.float32),
                   jax.ShapeDtypeStruct((G, 128), jnp.float32)],
    )(h, batch2, batchc)


def _mlp_body(zsum_ref, zmax_ref, cnt_ref, w1_ref, b1_ref, w2_ref, b2_ref,
              o_ref):
    cnt = jnp.maximum(cnt_ref[:, :1], 1.0)
    z = jnp.concatenate([zsum_ref[...] / cnt, zmax_ref[...]], axis=1)
    y = jnp.dot(z, w1_ref[...], preferred_element_type=jnp.float32) + b1_ref[...]
    y = jnp.maximum(y, 0.0)
    o_ref[...] = jnp.dot(y, w2_ref[...],
                         preferred_element_type=jnp.float32) + b2_ref[...]


def _mlp_call(zsum, zmax, cnt, w1, b1, w2, b2):
    return pl.pallas_call(
        _mlp_body,
        out_shape=jax.ShapeDtypeStruct((G, DOUT), jnp.float32),
    )(zsum, zmax, cnt, w1, b1, w2, b2)


def kernel(x, edge_index, batch, W0, b0, g0, be0, W1, b1, g1, be1,
           W2, b2, g2, be2, fc1_W, fc1_b, fc2_W, fc2_b):
    src = edge_index[0]
    dst = edge_index[1]
    dst_deg = dst.reshape(NC, NT, CH_D, K)
    batch2 = batch.reshape(NB, 1, RB)
    batchc = batch.reshape(N, 1)
    lo_tab = jnp.array([[0] * 16, [NH] * 16], jnp.int32)

    key = (dst >= NH).astype(jnp.int32)
    pos0 = jnp.cumsum(1 - key) - 1
    pos1 = jnp.arange(E, dtype=jnp.int32) - pos0 - 1
    e0 = pos0[-1] + 1
    e1 = E - e0
    CAP = NT * CH_P * K
    PU = NT * K * NBUF

    def _lay(slot):
        cg = slot // K
        return ((cg % NT) * CH_P + cg // NT) * K + slot % K

    p0 = jnp.where(key == 0, _lay(pos0), CAP)
    p1 = jnp.where(key == 1, _lay(pos1), CAP)
    srch = jnp.stack([
        jnp.zeros((CAP,), jnp.int32).at[p0].set(src, mode="drop",
                                                unique_indices=True),
        jnp.zeros((CAP,), jnp.int32).at[p1].set(src, mode="drop",
                                                unique_indices=True),
    ]).reshape(2, NT, CH_P, K)
    dsth = jnp.stack([
        jnp.full((CAP,), N, jnp.int32).at[p0].set(dst, mode="drop",
                                                  unique_indices=True),
        jnp.full((CAP,), N, jnp.int32).at[p1].set(dst, mode="drop",
                                                  unique_indices=True),
    ]).reshape(2, NT, CH_P, K)
    nch0 = NBUF * ((e0 + PU - 1) // PU)
    nch1 = NBUF * ((e1 + PU - 1) // PU)
    nch_tab = jnp.stack([jnp.full((16,), nch0, jnp.int32),
                         jnp.full((16,), nch1, jnp.int32)])

    deg2 = _deg_call(dst_deg).T

    za, zb = _mm0_call(deg2, x, W0)

    w_stack = jnp.stack([W1, W2, jnp.eye(H, dtype=jnp.float32)])
    g_stack = jnp.stack([g0, g1, g2])
    be_stack = jnp.stack([be0, be1, be2])

    def _layer(carry, wgb):
        za, zb, _ = carry
        w, g, be = wgb

        def _hprop(_, xs):
            zi0, zi1, lov, nchv, sr, dr = xs
            o0, o1 = _prop_call(za, zb, zi0, zi1, lov, nchv, sr, dr)
            return 0, (o0, o1)

        _, (o0s, o1s) = lax.scan(
            _hprop, 0,
            (za.reshape(2, NH, HH), zb.reshape(2, NH, HH), lo_tab,
             nch_tab, srch, dsth))
        sa = o0s.reshape(N, HH)
        sb = o1s.reshape(N, HH)
        stats = _stats_call(deg2, sa, sb)
        za2, zb2, h = _bnmm_call(deg2, sa, sb, stats, g, be, w)
        return (za2, zb2, h), None

    zero_h = jnp.zeros((N, H), jnp.float32)
    (_, _, h), _ = lax.scan(
        _layer, (za, zb, zero_h), (w_stack, g_stack, be_stack))

    zsum, zmax, cnt = _pool_call(h, batch2, batchc)
    return _mlp_call(zsum, zmax, cnt, fc1_W, fc1_b, fc2_W, fc2_b)


# speedup vs baseline: 3.4616x; 3.4614x over previous
import jax
import jax.numpy as jnp
from jax import lax
from jax.experimental import pallas as pl
from jax.experimental.pallas import tpu as pltpu
from jax.experimental.pallas import tpu_sc as plsc

N = 10000
E = 320000
DIN = 128
H = 256
HH = 128
G = 64
DOUT = 4
NP = 10240
NT = 16
NC = 2
K = 80
NBUF = 2
CH_P = E // NT // K
CH_D = E // (NC * NT) // K
NH = N // 2
DUMMY = NH
RPH = 312
HTAIL0 = RPH * NT
HTAILN = NH - HTAIL0
RPTP = NP // NT
RB = 1000
NB = N // RB


def _sc_mesh():
    return plsc.VectorSubcoreMesh(core_axis_name="c", subcore_axis_name="s")


def _deg_body(dst_hbm, out_hbm, idxbuf, ones_v, zeros_v, acc):
    c = lax.axis_index("c")
    s = lax.axis_index("s")

    def fz(i, _):
        zeros_v[pl.ds(i * 16, 16)] = jnp.zeros((16,), jnp.float32)
        return 0
    lax.fori_loop(0, RPTP // 16, fz, 0)

    def fo(i, _):
        ones_v[pl.ds(i * 16, 16)] = jnp.ones((16,), jnp.float32)
        return 0
    lax.fori_loop(0, K // 16, fo, 0)

    pltpu.sync_copy(zeros_v, acc.at[pl.ds(s * RPTP, RPTP)])
    pltpu.sync_copy(dst_hbm.at[c, s], idxbuf)
    plsc.subcore_barrier()

    def body(i, _):
        pltpu.sync_copy(ones_v, acc.at[idxbuf.at[i]], add=True)
        return 0
    lax.fori_loop(0, CH_D, body, 0)

    plsc.subcore_barrier()
    pltpu.sync_copy(acc.at[pl.ds(s * RPTP, RPTP)],
                    out_hbm.at[c, pl.ds(s * RPTP, RPTP)])


def _deg_call(dst_r):
    f = pl.kernel(
        _deg_body,
        out_type=jax.ShapeDtypeStruct((NC, NP), jnp.float32),
        mesh=_sc_mesh(),
        scratch_types=[
            pltpu.VMEM((CH_D, K), jnp.int32),
            pltpu.VMEM((K,), jnp.float32),
            pltpu.VMEM((RPTP,), jnp.float32),
            pltpu.VMEM_SHARED((NP,), jnp.float32),
        ],
    )
    return f(dst_r)


def _prop_body(zf0, zf1, zi0, zi1, lo_hbm, src_r, dst_r, out0, out1,
               sidx, didx, r0, r1, lobuf, acc, g0s, g1s, s0s, s1s):
    c = lax.axis_index("c")
    s = lax.axis_index("s")
    rowsb = (r0, r1)
    gsems = (g0s, g1s)
    ssems = (s0s, s1s)
    pltpu.sync_copy(lo_hbm, lobuf)
    lo = lobuf[...]
    for cc in (0, 1):
        zf = (zf0, zf1)[cc]
        zi = (zi0, zi1)[cc]
        out = (out0, out1)[cc]

        @pl.when(c == cc)
        def _(zf=zf, zi=zi, out=out):
            pltpu.sync_copy(zi.at[pl.ds(s * RPH, RPH)],
                            acc.at[pl.ds(s * RPH, RPH)])

            @pl.when(s == NT - 1)
            def _(zi=zi):
                pltpu.sync_copy(zi.at[pl.ds(HTAIL0, HTAILN)],
                                acc.at[pl.ds(HTAIL0, HTAILN)])

            pltpu.sync_copy(src_r.at[s], sidx)
            pltpu.sync_copy(dst_r.at[s], didx)
            plsc.subcore_barrier()

            def transform(i):
                for u in range(K // 16):
                    d = didx[i, pl.ds(u * 16, 16)]
                    m = (d >= lo) & (d < lo + NH)
                    didx[i, pl.ds(u * 16, 16)] = jnp.where(
                        m, d - lo, DUMMY + (d & 7))

            def start_gather(i, b, zf=zf):
                pltpu.async_copy(zf.at[sidx.at[i]], rowsb[b], gsems[b])

            def wait_gather(i, b, zf=zf):
                pltpu.make_async_copy(zf.at[sidx.at[i]], rowsb[b],
                                      gsems[b]).wait()

            def start_scatter(i, b):
                pltpu.async_copy(rowsb[b], acc.at[didx.at[i]], ssems[b],
                                 add=True)

            def wait_scatter(i, b):
                pltpu.make_async_copy(rowsb[b], acc.at[didx.at[i]],
                                      ssems[b]).wait()

            for b in range(NBUF - 1):
                start_gather(b, b)

            def body(j, _):
                for u in range(NBUF):
                    i = NBUF * j + u
                    b = u
                    transform(i)
                    wait_gather(i, b)
                    start_scatter(i, b)
                    ip = i + NBUF - 1
                    bp = (u + NBUF - 1) % NBUF

                    @pl.when(ip < CH_P)
                    def _(i=i, ip=ip, bp=bp):
                        @pl.when(i >= 1)
                        def _():
                            wait_scatter(i - 1, bp)
                        start_gather(ip, bp)
                return 0
            lax.fori_loop(0, CH_P // NBUF, body, 0)
            for i in range(CH_P - NBUF, CH_P):
                wait_scatter(i, i % NBUF)

            plsc.subcore_barrier()
            pltpu.sync_copy(acc.at[pl.ds(s * RPH, RPH)],
                            out.at[pl.ds(s * RPH, RPH)])

            @pl.when(s == NT - 1)
            def _(out=out):
                pltpu.sync_copy(acc.at[pl.ds(HTAIL0, HTAILN)],
                                out.at[pl.ds(HTAIL0, HTAILN)])


def _prop_call(zf0, zf1, zi0, zi1, lov, src_r, dst_r):
    f = pl.kernel(
        _prop_body,
        out_type=(jax.ShapeDtypeStruct((NH, HH), jnp.float32),
                  jax.ShapeDtypeStruct((NH, HH), jnp.float32)),
        mesh=_sc_mesh(),
        scratch_types=(
            [pltpu.VMEM((CH_P, K), jnp.int32),
             pltpu.VMEM((CH_P, K), jnp.int32)]
            + [pltpu.VMEM((K, HH), jnp.float32) for _ in range(NBUF)]
            + [pltpu.VMEM((16,), jnp.int32),
               pltpu.VMEM_SHARED((NH + 8, HH), jnp.float32)]
            + [pltpu.SemaphoreType.DMA for _ in range(2 * NBUF)]
        ),
    )
    return f(zf0, zf1, zi0, zi1, lov, src_r, dst_r)


def _dinv_of(deg_blk):
    deg = deg_blk[:, 0] + deg_blk[:, 1] + 1.0
    return lax.rsqrt(jnp.maximum(deg, 1e-12))


def _unhalf(o0_ref, o1_ref):
    return jnp.concatenate([o0_ref[...], o1_ref[...]], axis=1)


def _mm0_body(deg_ref, x_ref, w_ref, za_ref, zb_ref):
    dinv = _dinv_of(deg_ref[...])
    z = jnp.dot(x_ref[...], w_ref[...],
                preferred_element_type=jnp.float32) * dinv[:, None]
    za_ref[...] = z[:, :HH]
    zb_ref[...] = z[:, HH:]


def _mm0_call(deg2, x, w):
    return pl.pallas_call(
        _mm0_body,
        grid=(NB,),
        in_specs=[
            pl.BlockSpec((RB, NC), lambda i: (i, 0)),
            pl.BlockSpec((RB, DIN), lambda i: (i, 0)),
            pl.BlockSpec((DIN, H), lambda i: (0, 0)),
        ],
        out_specs=[
            pl.BlockSpec((RB, HH), lambda i: (i, 0)),
            pl.BlockSpec((RB, HH), lambda i: (i, 0)),
        ],
        out_shape=[jax.ShapeDtypeStruct((N, HH), jnp.float32),
                   jax.ShapeDtypeStruct((N, HH), jnp.float32)],
    )(deg2, x, w)


def _half_spec():
    return pl.BlockSpec((RB, HH), lambda i: (i, 0))


def _stats_body(deg_ref, o0_ref, o1_ref, o_ref):
    i = pl.program_id(0)
    dinv = _dinv_of(deg_ref[...])
    t = _unhalf(o0_ref, o1_ref) * dinv[:, None]
    part = jnp.stack([jnp.sum(t, axis=0), jnp.sum(t * t, axis=0)])

    @pl.when(i == 0)
    def _():
        o_ref[...] = part

    @pl.when(i > 0)
    def _():
        o_ref[...] = o_ref[...] + part


def _stats_call(deg2, o0s, o1s):
    return pl.pallas_call(
        _stats_body,
        grid=(NB,),
        in_specs=[
            pl.BlockSpec((RB, NC), lambda i: (i, 0)),
            _half_spec(),
            _half_spec(),
        ],
        out_specs=pl.BlockSpec((2, H), lambda i: (0, 0)),
        out_shape=jax.ShapeDtypeStruct((2, H), jnp.float32),
    )(deg2, o0s, o1s)


def _bnmm_body(deg_ref, o0_ref, o1_ref, stats_ref, g_ref, be_ref, w_ref,
               za_ref, zb_ref, h_ref):
    dinv = _dinv_of(deg_ref[...])
    t = _unhalf(o0_ref, o1_ref) * dinv[:, None]
    mean = stats_ref[0] / float(N)
    var = stats_ref[1] / float(N) - mean * mean
    h = (t - mean) * lax.rsqrt(var + 1e-5) * g_ref[...] + be_ref[...]
    h = jnp.maximum(h, 0.0)
    z = jnp.dot(h, w_ref[...], preferred_element_type=jnp.float32) * dinv[:, None]
    za_ref[...] = z[:, :HH]
    zb_ref[...] = z[:, HH:]
    h_ref[...] = h


def _bnmm_call(deg2, o0s, o1s, stats, g, be, w):
    return pl.pallas_call(
        _bnmm_body,
        grid=(NB,),
        in_specs=[
            pl.BlockSpec((RB, NC), lambda i: (i, 0)),
            _half_spec(),
            _half_spec(),
            pl.BlockSpec((2, H), lambda i: (0, 0)),
            pl.BlockSpec((H,), lambda i: (0,)),
            pl.BlockSpec((H,), lambda i: (0,)),
            pl.BlockSpec((H, H), lambda i: (0, 0)),
        ],
        out_specs=[
            pl.BlockSpec((RB, HH), lambda i: (i, 0)),
            pl.BlockSpec((RB, HH), lambda i: (i, 0)),
            pl.BlockSpec((RB, H), lambda i: (i, 0)),
        ],
        out_shape=[jax.ShapeDtypeStruct((N, HH), jnp.float32),
                   jax.ShapeDtypeStruct((N, HH), jnp.float32),
                   jax.ShapeDtypeStruct((N, H), jnp.float32)],
    )(deg2, o0s, o1s, stats, g, be, w)


def _pool_body(h_ref, bs_ref, bv_ref, zsum_ref, zmax_ref, cnt_ref):
    i = pl.program_id(0)

    @pl.when(i == 0)
    def _():
        zsum_ref[...] = jnp.zeros_like(zsum_ref)
        zmax_ref[...] = jnp.full_like(zmax_ref, -jnp.inf)
        cnt_ref[...] = jnp.zeros_like(cnt_ref)

    h = h_ref[...]
    bv = bv_ref[...]
    glo = bs_ref[0, 0, 0]
    ghi = bs_ref[0, 0, RB - 1]

    def body(gid, _):
        m = bv == gid
        hmax = jnp.max(jnp.where(m, h, -jnp.inf), axis=0)
        hsum = jnp.sum(jnp.where(m, h, 0.0), axis=0)
        cnt = jnp.sum(m.astype(jnp.float32))
        zmax_ref[pl.ds(gid, 1), :] = jnp.maximum(zmax_ref[pl.ds(gid, 1), :],
                                                 hmax[None])
        zsum_ref[pl.ds(gid, 1), :] = zsum_ref[pl.ds(gid, 1), :] + hsum[None]
        cnt_ref[pl.ds(gid, 1), :] = cnt_ref[pl.ds(gid, 1), :] + cnt
        return 0
    lax.fori_loop(glo, ghi + 1, body, 0)


def _pool_call(h, batch2, batchc):
    return pl.pallas_call(
        _pool_body,
        grid=(NB,),
        in_specs=[
            pl.BlockSpec((RB, H), lambda i: (i, 0)),
            pl.BlockSpec((1, 1, RB), lambda i: (i, 0, 0), memory_space=pltpu.SMEM),
            pl.BlockSpec((RB, 1), lambda i: (i, 0)),
        ],
        out_specs=[
            pl.BlockSpec((G, H), lambda i: (0, 0)),
            pl.BlockSpec((G, H), lambda i: (0, 0)),
            pl.BlockSpec((G, 128), lambda i: (0, 0)),
        ],
        out_shape=[jax.ShapeDtypeStruct((G, H), jnp.float32),
                   jax.ShapeDtypeStruct((G, H), jnp.float32),
                   jax.ShapeDtypeStruct((G, 128), jnp.float32)],
    )(h, batch2, batchc)


def _mlp_body(zsum_ref, zmax_ref, cnt_ref, w1_ref, b1_ref, w2_ref, b2_ref,
              o_ref):
    cnt = jnp.maximum(cnt_ref[:, :1], 1.0)
    z = jnp.concatenate([zsum_ref[...] / cnt, zmax_ref[...]], axis=1)
    y = jnp.dot(z, w1_ref[...], preferred_element_type=jnp.float32) + b1_ref[...]
    y = jnp.maximum(y, 0.0)
    o_ref[...] = jnp.dot(y, w2_ref[...],
                         preferred_element_type=jnp.float32) + b2_ref[...]


def _mlp_call(zsum, zmax, cnt, w1, b1, w2, b2):
    return pl.pallas_call(
        _mlp_body,
        out_shape=jax.ShapeDtypeStruct((G, DOUT), jnp.float32),
    )(zsum, zmax, cnt, w1, b1, w2, b2)


def kernel(x, edge_index, batch, W0, b0, g0, be0, W1, b1, g1, be1,
           W2, b2, g2, be2, fc1_W, fc1_b, fc2_W, fc2_b):
    src = edge_index[0]
    dst = edge_index[1]
    dst_deg = dst.reshape(NC, NT, CH_D, K)
    src_r = src.reshape(NT, CH_P, K)
    dst_r = dst.reshape(NT, CH_P, K)
    batch2 = batch.reshape(NB, 1, RB)
    batchc = batch.reshape(N, 1)
    lo_tab = jnp.array([[0] * 16, [NH] * 16], jnp.int32)

    deg2 = _deg_call(dst_deg).T

    za, zb = _mm0_call(deg2, x, W0)

    w_stack = jnp.stack([W1, W2, jnp.eye(H, dtype=jnp.float32)])
    g_stack = jnp.stack([g0, g1, g2])
    be_stack = jnp.stack([be0, be1, be2])

    def _layer(carry, wgb):
        za, zb, _ = carry
        w, g, be = wgb

        def _hprop(_, xs):
            zi0, zi1, lov = xs
            o0, o1 = _prop_call(za, zb, zi0, zi1, lov, src_r, dst_r)
            return 0, (o0, o1)

        _, (o0s, o1s) = lax.scan(
            _hprop, 0,
            (za.reshape(2, NH, HH), zb.reshape(2, NH, HH), lo_tab))
        sa = o0s.reshape(N, HH)
        sb = o1s.reshape(N, HH)
        stats = _stats_call(deg2, sa, sb)
        za2, zb2, h = _bnmm_call(deg2, sa, sb, stats, g, be, w)
        return (za2, zb2, h), None

    zero_h = jnp.zeros((N, H), jnp.float32)
    (_, _, h), _ = lax.scan(
        _layer, (za, zb, zero_h), (w_stack, g_stack, be_stack))

    zsum, zmax, cnt = _pool_call(h, batch2, batchc)
    return _mlp_call(zsum, zmax, cnt, fc1_W, fc1_b, fc2_W, fc2_b)
